# baseline (speedup 1.0000x reference)
import functools

import jax
import jax.numpy as jnp
from jax import lax
from jax.experimental import pallas as pl
from jax.experimental.pallas import tpu as pltpu
from jax.experimental.pallas import tpu_sc as plsc

B = 1024
L = 20
D = 64
V = 100000

NC = 2
NS = 16
NW = NC * NS
NQ = 4
NG = NW // NQ
DG = D // NG
Q_SPLIT = 25088
CHUNK = 12544
TAIL_MAIN = 12160
TAIL_T0 = V - 128
TAIL_CUT = V - 32
B_GROUPS = B // 16

N_TILE = 6144
N_STEPS = (V + N_TILE - 1) // N_TILE


def _sc_gather_mean_body(ids_t_hbm, table_t_hbm, tail_t_hbm, out_hbm, ids_v, buf_v, acc_v, part_hbm, sem):
    cid = lax.axis_index("c")
    sid = lax.axis_index("s")
    wid = cid * NS + sid
    q = wid % NQ
    grp = wid // NQ
    rb = pl.multiple_of(grp * DG, 8)
    ca = pl.multiple_of(q * Q_SPLIT, 128)

    for chunk in range(2):
        if chunk == 0:
            start = ca
            pltpu.async_copy(
                table_t_hbm.at[pl.ds(rb, DG), pl.ds(start, CHUNK)], buf_v, sem
            ).wait()
            end = start + CHUNK
        else:
            start = pl.multiple_of(ca + CHUNK, 128)

            @pl.when(q < NQ - 1)
            def _full():
                pltpu.async_copy(
                    table_t_hbm.at[pl.ds(rb, DG), pl.ds(start, CHUNK)], buf_v, sem
                ).wait()

            @pl.when(q == NQ - 1)
            def _tail():
                pltpu.async_copy(
                    table_t_hbm.at[pl.ds(rb, DG), pl.ds(start, TAIL_MAIN)],
                    buf_v.at[:, pl.ds(0, TAIL_MAIN)],
                    sem,
                ).wait()
                pltpu.async_copy(
                    tail_t_hbm.at[pl.ds(rb, DG)],
                    buf_v.at[:, pl.ds(TAIL_MAIN, 128)],
                    sem,
                ).wait()

            end = jnp.minimum(start + CHUNK, V)

        for bh in range(2):
            hb = bh * (B // 2)
            pltpu.sync_copy(ids_t_hbm.at[:, pl.ds(hb, B // 2)], ids_v)

            def group(g, carry):
                b0 = g * 16
                cols = []
                masks = []
                for j in range(L):
                    idx = ids_v[j, pl.ds(b0, 16)]
                    c = idx - start + jnp.where(idx >= TAIL_CUT, 96, 0)
                    inb = (c >= 0) & (idx < end)
                    cols.append(jnp.where(inb, c, 0))
                    masks.append(inb)
                for dl in range(DG):
                    row = jnp.full((16,), dl, jnp.int32)
                    if chunk == 0:
                        s = jnp.zeros((16,), jnp.float32)
                    else:
                        s = acc_v[dl, pl.ds(hb + b0, 16)]
                    for j in range(L):
                        v = plsc.load_gather(buf_v, [row, cols[j]])
                        s = s + jnp.where(masks[j], v, 0.0)
                    acc_v[dl, pl.ds(hb + b0, 16)] = s
                return carry

            lax.fori_loop(0, B_GROUPS // 2, group, 0)

    pltpu.sync_copy(acc_v, part_hbm.at[wid])
    plsc.subcore_barrier()

    @pl.when(sid % NQ == 0)
    def _combine():
        for t in range(1, NQ):
            pltpu.sync_copy(part_hbm.at[wid + t], buf_v.at[:, pl.ds(0, B)])

            def addg(g, carry):
                b0 = g * 16
                for dl in range(DG):
                    acc_v[dl, pl.ds(b0, 16)] = (
                        acc_v[dl, pl.ds(b0, 16)] + buf_v[dl, pl.ds(b0, 16)]
                    )
                return carry

            lax.fori_loop(0, B_GROUPS, addg, 0)

    @pl.when(sid % NQ == 0)
    def _emit():
        def scaleg(g, carry):
            b0 = g * 16
            for dl in range(DG):
                acc_v[dl, pl.ds(b0, 16)] = acc_v[dl, pl.ds(b0, 16)] * (1.0 / L)
            return carry

        lax.fori_loop(0, B_GROUPS, scaleg, 0)
        pltpu.sync_copy(acc_v, out_hbm.at[pl.ds(rb, DG)])


_sc_gather_mean = functools.partial(
    pl.kernel,
    mesh=plsc.VectorSubcoreMesh(core_axis_name="c", subcore_axis_name="s"),
    out_type=jax.ShapeDtypeStruct((D, B), jnp.float32),
    compiler_params=pltpu.CompilerParams(needs_layout_passes=False),
    scratch_types=[
        pltpu.VMEM((L, B // 2), jnp.int32),
        pltpu.VMEM((DG, CHUNK), jnp.float32),
        pltpu.VMEM((DG, B), jnp.float32),
        pltpu.HBM((NW, DG, B), jnp.float32),
        pltpu.SemaphoreType.DMA,
    ],
)(_sc_gather_mean_body)


def _mm_body(w_ref, x_ref, o_ref):
    o_ref[...] = lax.dot_general(
        w_ref[...],
        x_ref[...],
        dimension_numbers=(((0,), (0,)), ((), ())),
        preferred_element_type=jnp.float32,
    )


def kernel(context_ids, embedding_table, linear_weight):
    ids_t = context_ids.astype(jnp.int32).T
    table_t = embedding_table.T
    tail_t = table_t[:, TAIL_T0:]
    w_t = linear_weight.T
    ctx_mean_t = _sc_gather_mean(ids_t, table_t, tail_t)
    logits_t = pl.pallas_call(
        _mm_body,
        grid=(N_STEPS,),
        in_specs=[
            pl.BlockSpec((D, N_TILE), lambda n: (0, n)),
            pl.BlockSpec((D, B), lambda n: (0, 0)),
        ],
        out_specs=pl.BlockSpec((N_TILE, B), lambda n: (n, 0)),
        out_shape=jax.ShapeDtypeStruct((V, B), jnp.float32),
    )(w_t, ctx_mean_t)
    return logits_t.T


# speedup vs baseline: 1.0893x; 1.0893x over previous
import functools

import jax
import jax.numpy as jnp
from jax import lax
from jax.experimental import pallas as pl
from jax.experimental.pallas import tpu as pltpu
from jax.experimental.pallas import tpu_sc as plsc

B = 1024
L = 20
D = 64
V = 100000

NC = 2
NS = 16
NW = NC * NS
D_PER_W = D // NW
B_GROUPS = B // 16

N_TILE = 6144
N_STEPS = (V + N_TILE - 1) // N_TILE


def _sc_gather_mean_body(ids_t_hbm, table_t_hbm, out_hbm, ids_v, row_v, out_v, sem):
    wid = lax.axis_index("s") * NC + lax.axis_index("c")
    d0 = wid * D_PER_W

    pltpu.sync_copy(ids_t_hbm, ids_v)

    for k in range(D_PER_W):
        pltpu.async_copy(table_t_hbm.at[pl.ds((d0 + k) * V, V)], row_v, sem).wait()

        def group(g, carry):
            acc = jnp.zeros((16,), jnp.float32)
            for j in range(L):
                idx = ids_v[j, pl.ds(g * 16, 16)]
                acc = acc + plsc.load_gather(row_v, [idx])
            out_v[pl.ds(k * B + g * 16, 16)] = acc * (1.0 / L)
            return carry

        lax.fori_loop(0, B_GROUPS, group, 0)

    pltpu.sync_copy(out_v, out_hbm.at[pl.ds(d0 * B, D_PER_W * B)])


_sc_gather_mean = functools.partial(
    pl.kernel,
    mesh=plsc.VectorSubcoreMesh(core_axis_name="c", subcore_axis_name="s"),
    out_type=jax.ShapeDtypeStruct((D * B,), jnp.float32),
    compiler_params=pltpu.CompilerParams(needs_layout_passes=False),
    scratch_types=[
        pltpu.VMEM((L, B), jnp.int32),
        pltpu.VMEM((V,), jnp.float32),
        pltpu.VMEM((D_PER_W * B,), jnp.float32),
        pltpu.SemaphoreType.DMA,
    ],
)(_sc_gather_mean_body)


def _mm_body(w_ref, x_ref, o_ref):
    o_ref[...] = lax.dot_general(
        w_ref[...],
        x_ref[...],
        dimension_numbers=(((0,), (0,)), ((), ())),
        preferred_element_type=jnp.float32,
    )


def kernel(context_ids, embedding_table, linear_weight):
    ids_t = context_ids.astype(jnp.int32).T
    table_t = embedding_table.T.reshape(D * V)
    w_t = linear_weight.T
    ctx_mean_t = _sc_gather_mean(ids_t, table_t).reshape(D, B)
    logits_t = pl.pallas_call(
        _mm_body,
        grid=(N_STEPS,),
        in_specs=[
            pl.BlockSpec((D, N_TILE), lambda n: (0, n)),
            pl.BlockSpec((D, B), lambda n: (0, 0)),
        ],
        out_specs=pl.BlockSpec((N_TILE, B), lambda n: (n, 0)),
        out_shape=jax.ShapeDtypeStruct((V, B), jnp.float32),
    )(w_t, ctx_mean_t)
    return logits_t.T
